# baseline (speedup 1.0000x reference)
import functools

import numpy as np
import jax
import jax.numpy as jnp
from jax import lax
from jax.experimental import pallas as pl
from jax.experimental.pallas import tpu as pltpu

_NUM_CLASSES = 80
_MAX_BOXES = 20
_SCORE_THRESHOLD = 0.3
_IOU_THRESHOLD = 0.5
_ANCHOR_MASK = [[6, 7, 8], [3, 4, 5], [0, 1, 2]]
_GRIDS = [19, 38, 76]
_INPUT_DIM = 608.0

_N = 3 * sum(g * g for g in _GRIDS)
_NPAD = ((_N + 127) // 128) * 128


def _column_layout():
    gxs, gys, Gs, aidx = [], [], [], []
    for l, g in enumerate(_GRIDS):
        yy, xx = np.meshgrid(np.arange(g), np.arange(g), indexing="ij")
        gx = np.repeat(xx.reshape(-1), 3)
        gy = np.repeat(yy.reshape(-1), 3)
        gxs.append(gx)
        gys.append(gy)
        Gs.append(np.full(g * g * 3, g, dtype=np.float32))
        aidx.append(np.tile(np.array(_ANCHOR_MASK[l], dtype=np.int32), g * g))
    return (
        np.concatenate(gxs).astype(np.float32),
        np.concatenate(gys).astype(np.float32),
        np.concatenate(Gs),
        np.concatenate(aidx),
    )


_COL_GX, _COL_GY, _COL_G, _COL_AIDX = _column_layout()
_AIDX_ONEHOT = (_COL_AIDX[:, None] == np.arange(9)[None, :]).astype(np.float32)


def _sigmoid(x):
    return jax.nn.sigmoid(x)


def _nms_body(f_ref, meta_ref, boxes_out_ref, scores_out_ref, s_ref, bx_ref):
    npad = _NPAD
    fmat = jnp.transpose(f_ref[...], (1, 0))
    sx = _sigmoid(fmat[0:1, :])
    sy = _sigmoid(fmat[1:2, :])
    ew = jnp.exp(fmat[2:3, :])
    eh = jnp.exp(fmat[3:4, :])
    c1 = meta_ref[0:1, :]
    c2 = meta_ref[1:2, :]
    c3 = meta_ref[2:3, :]
    d1 = meta_ref[3:4, :]
    d2 = meta_ref[4:5, :]
    d3 = meta_ref[5:6, :]
    ycen = c1 * sy + c2
    xcen = d1 * sx + d2
    hh = c3 * eh
    ww = d3 * ew
    y0 = ycen - hh
    y1 = ycen + hh
    x0 = xcen - ww
    x1 = xcen + ww
    bx_ref[0:1, :] = y0
    bx_ref[1:2, :] = x0
    bx_ref[2:3, :] = y1
    bx_ref[3:4, :] = x1
    bx_ref[4:5, :] = (y1 - y0) * (x1 - x0)
    bx_ref[5:6, :] = jnp.zeros((1, npad), jnp.float32)
    bx_ref[6:7, :] = jnp.zeros((1, npad), jnp.float32)
    bx_ref[7:8, :] = jnp.zeros((1, npad), jnp.float32)

    conf = _sigmoid(fmat[4:5, :])
    s = _sigmoid(fmat[5:85, :]) * conf
    s_ref[...] = jnp.where(s >= _SCORE_THRESHOLD, s, -1.0)

    boxes_out_ref[...] = jnp.zeros((_NUM_CLASSES, 128), jnp.float32)
    scores_out_ref[...] = jnp.zeros((_NUM_CLASSES, 128), jnp.float32)

    iota = lax.broadcasted_iota(jnp.int32, (1, npad), 1)

    for t in range(_MAX_BOXES):
        s = s_ref[...]
        m = jnp.max(s, axis=1, keepdims=True)
        idx = jnp.min(jnp.where(s == m, iota, npad), axis=1, keepdims=True)
        oh = (iota == idx).astype(jnp.float32)
        b = lax.dot_general(
            oh, bx_ref[...],
            (((1,), (1,)), ((), ())),
            preferred_element_type=jnp.float32,
        )
        by0 = b[:, 0:1]
        bx0 = b[:, 1:2]
        by1 = b[:, 2:3]
        bx1 = b[:, 3:4]
        ba = b[:, 4:5]
        iy0 = jnp.maximum(by0, bx_ref[0:1, :])
        ix0 = jnp.maximum(bx0, bx_ref[1:2, :])
        iy1 = jnp.minimum(by1, bx_ref[2:3, :])
        ix1 = jnp.minimum(bx1, bx_ref[3:4, :])
        ih_ = jnp.maximum(iy1 - iy0, 0.0)
        iw_ = jnp.maximum(ix1 - ix0, 0.0)
        inter = ih_ * iw_
        kill = (inter + inter > (ba + 1e-9) + (bx_ref[4:5, :] - inter)) | (iota == idx)
        s_ref[...] = jnp.where(kill, -1.0, s)

        validf = (m > 0.0).astype(jnp.float32)
        scores_out_ref[:, t:t + 1] = m * validf
        boxes_out_ref[:, 4 * t:4 * t + 4] = b[:, 0:4] * validf


@functools.partial(jax.jit, static_argnames=())
def kernel(yolo_out0, yolo_out1, yolo_out2, input_image_shape, anchors):
    outs = (yolo_out0, yolo_out1, yolo_out2)
    rows = []
    for o in outs:
        gy, gx = o.shape[1], o.shape[2]
        rows.append(o.reshape(gy * gx * 3, 85))
    f = jnp.concatenate(rows, axis=0)
    fT = jnp.pad(f, ((0, _NPAD - _N), (0, 3)))

    image_shape = input_image_shape.astype(jnp.float32)
    input_shape = jnp.array([_INPUT_DIM, _INPUT_DIM], jnp.float32)
    new_shape = jnp.round(image_shape * jnp.min(input_shape / image_shape))
    offset = (input_shape - new_shape) / 2.0 / input_shape
    scale = input_shape / new_shape
    ih, iw = image_shape[0], image_shape[1]
    oy, ox = offset[0], offset[1]
    sy_, sx_ = scale[0], scale[1]

    aw_ah = jnp.asarray(_AIDX_ONEHOT) @ anchors
    aw = aw_ah[:, 0]
    ah = aw_ah[:, 1]
    G = jnp.asarray(_COL_G)
    c1 = ih * sy_ / G
    c2 = ih * sy_ * (jnp.asarray(_COL_GY) / G - oy)
    c3 = ih * sy_ * ah / (2.0 * _INPUT_DIM)
    d1 = iw * sx_ / G
    d2 = iw * sx_ * (jnp.asarray(_COL_GX) / G - ox)
    d3 = iw * sx_ * aw / (2.0 * _INPUT_DIM)
    meta = jnp.stack([c1, c2, c3, d1, d2, d3,
                      jnp.zeros_like(c1), jnp.zeros_like(c1)])
    meta = jnp.pad(meta, ((0, 0), (0, _NPAD - _N)))

    boxes_out, scores_out = pl.pallas_call(
        _nms_body,
        out_shape=[
            jax.ShapeDtypeStruct((_NUM_CLASSES, 128), jnp.float32),
            jax.ShapeDtypeStruct((_NUM_CLASSES, 128), jnp.float32),
        ],
        scratch_shapes=[
            pltpu.VMEM((_NUM_CLASSES, _NPAD), jnp.float32),
            pltpu.VMEM((8, _NPAD), jnp.float32),
        ],
    )(fT, meta)

    sel_boxes = boxes_out[:, : 4 * _MAX_BOXES].reshape(-1, 4)
    sel_scores = scores_out[:, :_MAX_BOXES].reshape(-1)
    classes = jnp.broadcast_to(
        jnp.arange(_NUM_CLASSES, dtype=jnp.int32)[:, None],
        (_NUM_CLASSES, _MAX_BOXES),
    ).reshape(-1)
    return sel_boxes, sel_scores, classes


# speedup vs baseline: 1.0799x; 1.0799x over previous
import functools

import numpy as np
import jax
import jax.numpy as jnp
from jax import lax
from jax.experimental import pallas as pl
from jax.experimental.pallas import tpu as pltpu

_NUM_CLASSES = 80
_MAX_BOXES = 20
_SCORE_THRESHOLD = 0.3
_IOU_THRESHOLD = 0.5
_ANCHOR_MASK = [[6, 7, 8], [3, 4, 5], [0, 1, 2]]
_GRIDS = [19, 38, 76]
_INPUT_DIM = 608.0

_N = 3 * sum(g * g for g in _GRIDS)
_NPAD = ((_N + 127) // 128) * 128


def _column_layout():
    gxs, gys, Gs, aidx = [], [], [], []
    for l, g in enumerate(_GRIDS):
        yy, xx = np.meshgrid(np.arange(g), np.arange(g), indexing="ij")
        gx = np.repeat(xx.reshape(-1), 3)
        gy = np.repeat(yy.reshape(-1), 3)
        gxs.append(gx)
        gys.append(gy)
        Gs.append(np.full(g * g * 3, g, dtype=np.float32))
        aidx.append(np.tile(np.array(_ANCHOR_MASK[l], dtype=np.int32), g * g))
    return (
        np.concatenate(gxs).astype(np.float32),
        np.concatenate(gys).astype(np.float32),
        np.concatenate(Gs),
        np.concatenate(aidx),
    )


_COL_GX, _COL_GY, _COL_G, _COL_AIDX = _column_layout()
_AIDX_ONEHOT = (_COL_AIDX[:, None] == np.arange(9)[None, :]).astype(np.float32)


def _sigmoid(x):
    return jax.nn.sigmoid(x)


def _nms_body(f_ref, meta_ref, boxes_out_ref, scores_out_ref, s_ref, bx_ref):
    npad = _NPAD
    sx = _sigmoid(f_ref[0:1, :])
    sy = _sigmoid(f_ref[1:2, :])
    ew = jnp.exp(f_ref[2:3, :])
    eh = jnp.exp(f_ref[3:4, :])
    c1 = meta_ref[0:1, :]
    c2 = meta_ref[1:2, :]
    c3 = meta_ref[2:3, :]
    d1 = meta_ref[3:4, :]
    d2 = meta_ref[4:5, :]
    d3 = meta_ref[5:6, :]
    ycen = c1 * sy + c2
    xcen = d1 * sx + d2
    hh = c3 * eh
    ww = d3 * ew
    y0 = ycen - hh
    y1 = ycen + hh
    x0 = xcen - ww
    x1 = xcen + ww
    bx_ref[0:1, :] = y0
    bx_ref[1:2, :] = x0
    bx_ref[2:3, :] = y1
    bx_ref[3:4, :] = x1
    bx_ref[4:5, :] = (y1 - y0) * (x1 - x0)
    bx_ref[5:6, :] = jnp.zeros((1, npad), jnp.float32)
    bx_ref[6:7, :] = jnp.zeros((1, npad), jnp.float32)
    bx_ref[7:8, :] = jnp.zeros((1, npad), jnp.float32)

    conf = _sigmoid(f_ref[4:5, :])
    s = _sigmoid(f_ref[5:85, :]) * conf
    s_ref[...] = jnp.where(s >= _SCORE_THRESHOLD, s, -1.0)

    boxes_out_ref[...] = jnp.zeros((_NUM_CLASSES, 128), jnp.float32)
    scores_out_ref[...] = jnp.zeros((_NUM_CLASSES, 128), jnp.float32)

    iota = lax.broadcasted_iota(jnp.int32, (1, npad), 1)

    for t in range(_MAX_BOXES):
        s = s_ref[...]
        m = jnp.max(s, axis=1, keepdims=True)
        idx = jnp.min(jnp.where(s == m, iota, npad), axis=1, keepdims=True)
        oh = (iota == idx).astype(jnp.float32)
        b = lax.dot_general(
            oh, bx_ref[...],
            (((1,), (1,)), ((), ())),
            preferred_element_type=jnp.float32,
        )
        by0 = b[:, 0:1]
        bx0 = b[:, 1:2]
        by1 = b[:, 2:3]
        bx1 = b[:, 3:4]
        ba = b[:, 4:5]
        iy0 = jnp.maximum(by0, bx_ref[0:1, :])
        ix0 = jnp.maximum(bx0, bx_ref[1:2, :])
        iy1 = jnp.minimum(by1, bx_ref[2:3, :])
        ix1 = jnp.minimum(bx1, bx_ref[3:4, :])
        ih_ = jnp.maximum(iy1 - iy0, 0.0)
        iw_ = jnp.maximum(ix1 - ix0, 0.0)
        inter = ih_ * iw_
        kill = (3.0 * inter > (ba + 1e-9) + bx_ref[4:5, :]) | (iota == idx)
        s_ref[...] = jnp.where(kill, -1.0, s)

        validf = (m > 0.0).astype(jnp.float32)
        scores_out_ref[:, t:t + 1] = m * validf
        boxes_out_ref[:, 4 * t:4 * t + 4] = b[:, 0:4] * validf


@functools.partial(jax.jit, static_argnames=())
def kernel(yolo_out0, yolo_out1, yolo_out2, input_image_shape, anchors):
    outs = (yolo_out0, yolo_out1, yolo_out2)
    rows = []
    for o in outs:
        gy, gx = o.shape[1], o.shape[2]
        rows.append(o.reshape(gy * gx * 3, 85))
    f = jnp.concatenate(rows, axis=0).T
    fT = jnp.pad(f, ((0, 3), (0, _NPAD - _N)))

    image_shape = input_image_shape.astype(jnp.float32)
    input_shape = jnp.array([_INPUT_DIM, _INPUT_DIM], jnp.float32)
    new_shape = jnp.round(image_shape * jnp.min(input_shape / image_shape))
    offset = (input_shape - new_shape) / 2.0 / input_shape
    scale = input_shape / new_shape
    ih, iw = image_shape[0], image_shape[1]
    oy, ox = offset[0], offset[1]
    sy_, sx_ = scale[0], scale[1]

    aw_ah = jnp.asarray(_AIDX_ONEHOT) @ anchors
    aw = aw_ah[:, 0]
    ah = aw_ah[:, 1]
    G = jnp.asarray(_COL_G)
    c1 = ih * sy_ / G
    c2 = ih * sy_ * (jnp.asarray(_COL_GY) / G - oy)
    c3 = ih * sy_ * ah / (2.0 * _INPUT_DIM)
    d1 = iw * sx_ / G
    d2 = iw * sx_ * (jnp.asarray(_COL_GX) / G - ox)
    d3 = iw * sx_ * aw / (2.0 * _INPUT_DIM)
    meta = jnp.stack([c1, c2, c3, d1, d2, d3,
                      jnp.zeros_like(c1), jnp.zeros_like(c1)])
    meta = jnp.pad(meta, ((0, 0), (0, _NPAD - _N)))

    boxes_out, scores_out = pl.pallas_call(
        _nms_body,
        out_shape=[
            jax.ShapeDtypeStruct((_NUM_CLASSES, 128), jnp.float32),
            jax.ShapeDtypeStruct((_NUM_CLASSES, 128), jnp.float32),
        ],
        scratch_shapes=[
            pltpu.VMEM((_NUM_CLASSES, _NPAD), jnp.float32),
            pltpu.VMEM((8, _NPAD), jnp.float32),
        ],
    )(fT, meta)

    sel_boxes = boxes_out[:, : 4 * _MAX_BOXES].reshape(-1, 4)
    sel_scores = scores_out[:, :_MAX_BOXES].reshape(-1)
    classes = jnp.broadcast_to(
        jnp.arange(_NUM_CLASSES, dtype=jnp.int32)[:, None],
        (_NUM_CLASSES, _MAX_BOXES),
    ).reshape(-1)
    return sel_boxes, sel_scores, classes


# speedup vs baseline: 1.1025x; 1.0209x over previous
import functools

import numpy as np
import jax
import jax.numpy as jnp
from jax import lax
from jax.experimental import pallas as pl
from jax.experimental.pallas import tpu as pltpu

_NUM_CLASSES = 80
_MAX_BOXES = 20
_SCORE_THRESHOLD = 0.3
_IOU_THRESHOLD = 0.5
_ANCHOR_MASK = [[6, 7, 8], [3, 4, 5], [0, 1, 2]]
_GRIDS = [19, 38, 76]
_INPUT_DIM = 608.0

_N = 3 * sum(g * g for g in _GRIDS)
_NPAD = ((_N + 127) // 128) * 128


def _column_layout():
    gxs, gys, Gs, aidx = [], [], [], []
    for l, g in enumerate(_GRIDS):
        yy, xx = np.meshgrid(np.arange(g), np.arange(g), indexing="ij")
        gx = np.repeat(xx.reshape(-1), 3)
        gy = np.repeat(yy.reshape(-1), 3)
        gxs.append(gx)
        gys.append(gy)
        Gs.append(np.full(g * g * 3, g, dtype=np.float32))
        aidx.append(np.tile(np.array(_ANCHOR_MASK[l], dtype=np.int32), g * g))
    return (
        np.concatenate(gxs).astype(np.float32),
        np.concatenate(gys).astype(np.float32),
        np.concatenate(Gs),
        np.concatenate(aidx),
    )


_COL_GX, _COL_GY, _COL_G, _COL_AIDX = _column_layout()
_AIDX_ONEHOT = (_COL_AIDX[:, None] == np.arange(9)[None, :]).astype(np.float32)


def _sigmoid(x):
    return jax.nn.sigmoid(x)


def _nms_body(f_ref, meta_ref, boxes_out_ref, scores_out_ref, s_ref, bx_ref):
    npad = _NPAD
    sx = _sigmoid(f_ref[0:1, :])
    sy = _sigmoid(f_ref[1:2, :])
    ew = jnp.exp(f_ref[2:3, :])
    eh = jnp.exp(f_ref[3:4, :])
    c1 = meta_ref[0:1, :]
    c2 = meta_ref[1:2, :]
    c3 = meta_ref[2:3, :]
    d1 = meta_ref[3:4, :]
    d2 = meta_ref[4:5, :]
    d3 = meta_ref[5:6, :]
    ycen = c1 * sy + c2
    xcen = d1 * sx + d2
    hh = c3 * eh
    ww = d3 * ew
    y0 = ycen - hh
    y1 = ycen + hh
    x0 = xcen - ww
    x1 = xcen + ww
    bx_ref[0:1, :] = y0
    bx_ref[1:2, :] = x0
    bx_ref[2:3, :] = y1
    bx_ref[3:4, :] = x1
    bx_ref[4:5, :] = (y1 - y0) * (x1 - x0)
    bx_ref[5:6, :] = jnp.zeros((1, npad), jnp.float32)
    bx_ref[6:7, :] = jnp.zeros((1, npad), jnp.float32)
    bx_ref[7:8, :] = jnp.zeros((1, npad), jnp.float32)

    conf = _sigmoid(f_ref[4:5, :])
    s = _sigmoid(f_ref[5:85, :]) * conf
    s = jnp.where(s >= _SCORE_THRESHOLD, s, -1.0)

    boxes_out_ref[...] = jnp.zeros((_NUM_CLASSES, 128), jnp.float32)
    scores_out_ref[...] = jnp.zeros((_NUM_CLASSES, 128), jnp.float32)

    iota = lax.broadcasted_iota(jnp.int32, (1, npad), 1)

    for t in range(_MAX_BOXES):
        m = jnp.max(s, axis=1, keepdims=True)
        idx = jnp.min(jnp.where(s == m, iota, npad), axis=1, keepdims=True)
        oh = (iota == idx).astype(jnp.float32)
        b = lax.dot_general(
            oh, bx_ref[...],
            (((1,), (1,)), ((), ())),
            preferred_element_type=jnp.float32,
        )
        by0 = b[:, 0:1]
        bx0 = b[:, 1:2]
        by1 = b[:, 2:3]
        bx1 = b[:, 3:4]
        ba = b[:, 4:5]
        iy0 = jnp.maximum(by0, bx_ref[0:1, :])
        ix0 = jnp.maximum(bx0, bx_ref[1:2, :])
        iy1 = jnp.minimum(by1, bx_ref[2:3, :])
        ix1 = jnp.minimum(bx1, bx_ref[3:4, :])
        ih_ = jnp.maximum(iy1 - iy0, 0.0)
        iw_ = jnp.maximum(ix1 - ix0, 0.0)
        inter = ih_ * iw_
        kill = (3.0 * inter > (ba + 1e-9) + bx_ref[4:5, :]) | (iota == idx)
        s = jnp.where(kill, -1.0, s)

        validf = (m > 0.0).astype(jnp.float32)
        scores_out_ref[:, t:t + 1] = m * validf
        boxes_out_ref[:, 4 * t:4 * t + 4] = b[:, 0:4] * validf


@functools.partial(jax.jit, static_argnames=())
def kernel(yolo_out0, yolo_out1, yolo_out2, input_image_shape, anchors):
    outs = (yolo_out0, yolo_out1, yolo_out2)
    rows = []
    for o in outs:
        gy, gx = o.shape[1], o.shape[2]
        rows.append(o.reshape(gy * gx * 3, 85))
    f = jnp.concatenate(rows, axis=0).T
    fT = jnp.pad(f, ((0, 3), (0, _NPAD - _N)))

    image_shape = input_image_shape.astype(jnp.float32)
    input_shape = jnp.array([_INPUT_DIM, _INPUT_DIM], jnp.float32)
    new_shape = jnp.round(image_shape * jnp.min(input_shape / image_shape))
    offset = (input_shape - new_shape) / 2.0 / input_shape
    scale = input_shape / new_shape
    ih, iw = image_shape[0], image_shape[1]
    oy, ox = offset[0], offset[1]
    sy_, sx_ = scale[0], scale[1]

    aw_ah = jnp.asarray(_AIDX_ONEHOT) @ anchors
    aw = aw_ah[:, 0]
    ah = aw_ah[:, 1]
    G = jnp.asarray(_COL_G)
    c1 = ih * sy_ / G
    c2 = ih * sy_ * (jnp.asarray(_COL_GY) / G - oy)
    c3 = ih * sy_ * ah / (2.0 * _INPUT_DIM)
    d1 = iw * sx_ / G
    d2 = iw * sx_ * (jnp.asarray(_COL_GX) / G - ox)
    d3 = iw * sx_ * aw / (2.0 * _INPUT_DIM)
    meta = jnp.stack([c1, c2, c3, d1, d2, d3,
                      jnp.zeros_like(c1), jnp.zeros_like(c1)])
    meta = jnp.pad(meta, ((0, 0), (0, _NPAD - _N)))

    boxes_out, scores_out = pl.pallas_call(
        _nms_body,
        out_shape=[
            jax.ShapeDtypeStruct((_NUM_CLASSES, 128), jnp.float32),
            jax.ShapeDtypeStruct((_NUM_CLASSES, 128), jnp.float32),
        ],
        scratch_shapes=[
            pltpu.VMEM((_NUM_CLASSES, _NPAD), jnp.float32),
            pltpu.VMEM((8, _NPAD), jnp.float32),
        ],
    )(fT, meta)

    sel_boxes = boxes_out[:, : 4 * _MAX_BOXES].reshape(-1, 4)
    sel_scores = scores_out[:, :_MAX_BOXES].reshape(-1)
    classes = jnp.broadcast_to(
        jnp.arange(_NUM_CLASSES, dtype=jnp.int32)[:, None],
        (_NUM_CLASSES, _MAX_BOXES),
    ).reshape(-1)
    return sel_boxes, sel_scores, classes
